# Initial kernel scaffold; baseline (speedup 1.0000x reference)
import jax
import jax.numpy as jnp
from jax.experimental import pallas as pl


def kernel(actions, action_embed, learned_token):
    raise NotImplementedError("write your pallas kernel here")


# speedup vs baseline: 1.4086x; 1.4086x over previous
import functools

import jax
import jax.numpy as jnp
from jax import lax
from jax.experimental import pallas as pl
from jax.experimental.pallas import tpu as pltpu
from jax.experimental.pallas import tpu_sc as plsc

_D = 64
_NT = 2
_OUT_W = _NT * _D
_NC = 2
_NS = 16
_NW = _NC * _NS
_L = 16
_CH = 128


@functools.lru_cache(maxsize=None)
def _build_sc_call(n_total):
    n_per = n_total // _NW
    n_chunks = n_per // _CH
    mesh = plsc.VectorSubcoreMesh(
        core_axis_name="c", subcore_axis_name="s", num_cores=_NC,
        num_subcores=_NS)

    def body(idx_hbm, table_hbm, tok_hbm, out_hbm,
             idx_v, rows_v, out_v, tok_v, gsem):
        wid = lax.axis_index("s") * _NC + lax.axis_index("c")
        base = wid * n_per
        pltpu.sync_copy(idx_hbm.at[pl.ds(base, n_per)], idx_v)
        pltpu.sync_copy(tok_hbm, tok_v)
        t0 = [tok_v[0, pl.ds(g * _L, _L)] for g in range(_D // _L)]
        t1 = [tok_v[1, pl.ds(g * _L, _L)] for g in range(_D // _L)]

        def chunk_body(c, carry):
            pltpu.async_copy(
                table_hbm.at[idx_v.at[pl.ds(c * _CH, _CH)]], rows_v,
                gsem).wait()

            def row_body(r, carry2):
                for g in range(_D // _L):
                    v = rows_v[r, pl.ds(g * _L, _L)]
                    out_v[r, pl.ds(g * _L, _L)] = v + t0[g]
                    out_v[r, pl.ds(_D + g * _L, _L)] = v + t1[g]
                return carry2

            lax.fori_loop(0, _CH, row_body, 0)
            pltpu.sync_copy(out_v, out_hbm.at[pl.ds(base + c * _CH, _CH)])
            return carry

        lax.fori_loop(0, n_chunks, chunk_body, 0)

    return pl.kernel(
        body,
        out_type=jax.ShapeDtypeStruct((n_total, _OUT_W), jnp.float32),
        mesh=mesh,
        scratch_types=[
            pltpu.VMEM((n_per,), jnp.int32),
            pltpu.VMEM((_CH, _D), jnp.float32),
            pltpu.VMEM((_CH, _OUT_W), jnp.float32),
            pltpu.VMEM((_NT, _D), jnp.float32),
            pltpu.SemaphoreType.DMA,
        ],
        compiler_params=pltpu.CompilerParams(use_tc_tiling_on_sc=False),
    )


def kernel(actions, action_embed, learned_token):
    b, t = actions.shape
    idx = actions.reshape(b * t).astype(jnp.int32)
    tok = learned_token.reshape(_NT, _D)
    out = _build_sc_call(b * t)(idx, action_embed, tok)
    return out.reshape(b, t, _NT, _D)


# speedup vs baseline: 1.5788x; 1.1208x over previous
import functools

import jax
import jax.numpy as jnp
from jax import lax
from jax.experimental import pallas as pl
from jax.experimental.pallas import tpu as pltpu
from jax.experimental.pallas import tpu_sc as plsc

_D = 64
_NT = 2
_OUT_W = _NT * _D
_NC = 2
_NS = 16
_NW = _NC * _NS
_CH = 128
_BUILD_ROWS = 2000


def _build_table2_body(tok_ref, tab_ref, out_ref):
    rows = tab_ref[...]
    out_ref[:, :_D] = rows + tok_ref[0:1, :]
    out_ref[:, _D:] = rows + tok_ref[1:2, :]


@functools.lru_cache(maxsize=None)
def _build_table2_call(num_rows):
    grid = num_rows // _BUILD_ROWS
    return pl.pallas_call(
        _build_table2_body,
        grid=(grid,),
        in_specs=[
            pl.BlockSpec((_NT, _D), lambda i: (0, 0)),
            pl.BlockSpec((_BUILD_ROWS, _D), lambda i: (i, 0)),
        ],
        out_specs=pl.BlockSpec((_BUILD_ROWS, _OUT_W), lambda i: (i, 0)),
        out_shape=jax.ShapeDtypeStruct((num_rows, _OUT_W), jnp.float32),
    )


@functools.lru_cache(maxsize=None)
def _gather_call(n_total):
    n_per = n_total // _NW
    n_chunks = n_per // _CH
    mesh = plsc.VectorSubcoreMesh(
        core_axis_name="c", subcore_axis_name="s", num_cores=_NC,
        num_subcores=_NS)

    def body(idx_hbm, tab2_hbm, out_hbm, idx_v, buf0, buf1, gsem0, gsem1):
        wid = lax.axis_index("s") * _NC + lax.axis_index("c")
        base = wid * n_per
        pltpu.sync_copy(idx_hbm.at[pl.ds(base, n_per)], idx_v)
        bufs = (buf0, buf1)
        gsems = (gsem0, gsem1)

        pltpu.async_copy(
            tab2_hbm.at[idx_v.at[pl.ds(0, _CH)]], buf0, gsem0)

        def pair_body(i, carry):
            for k in range(2):
                c = 2 * i + k
                bc, bn = bufs[k], bufs[1 - k]
                sc, sn = gsems[k], gsems[1 - k]
                pltpu.make_async_copy(
                    tab2_hbm.at[pl.ds(0, _CH)], bc, sc).wait()

                @pl.when(c + 1 < n_chunks)
                def _():
                    pltpu.async_copy(
                        tab2_hbm.at[idx_v.at[pl.ds((c + 1) * _CH, _CH)]],
                        bn, sn)

                pltpu.sync_copy(bc, out_hbm.at[pl.ds(base + c * _CH, _CH)])
            return carry

        lax.fori_loop(0, n_chunks // 2, pair_body, 0)

    return pl.kernel(
        body,
        out_type=jax.ShapeDtypeStruct((n_total, _OUT_W), jnp.float32),
        mesh=mesh,
        scratch_types=[
            pltpu.VMEM((n_per,), jnp.int32),
            pltpu.VMEM((_CH, _OUT_W), jnp.float32),
            pltpu.VMEM((_CH, _OUT_W), jnp.float32),
            pltpu.SemaphoreType.DMA,
            pltpu.SemaphoreType.DMA,
        ],
    )


def kernel(actions, action_embed, learned_token):
    b, t = actions.shape
    idx = actions.reshape(b * t).astype(jnp.int32)
    tok = learned_token.reshape(_NT, _D)
    table2 = _build_table2_call(action_embed.shape[0])(tok, action_embed)
    out = _gather_call(b * t)(idx, table2)
    return out.reshape(b, t, _NT, _D)


# speedup vs baseline: 1.7046x; 1.0797x over previous
import functools

import jax
import jax.numpy as jnp
from jax import lax
from jax.experimental import pallas as pl
from jax.experimental.pallas import tpu as pltpu
from jax.experimental.pallas import tpu_sc as plsc

_D = 64
_NT = 2
_OUT_W = _NT * _D
_NC = 2
_NS = 16
_NW = _NC * _NS
_L = 16
_BB = 128
_C_BUILD = 1024


def _build_table2_body(tok_ref, tabT_ref, out_ref):
    rows = tabT_ref[...].T
    out_ref[:, :_D] = rows + tok_ref[0:1, :]
    out_ref[:, _D:] = rows + tok_ref[1:2, :]


@functools.lru_cache(maxsize=None)
def _build_table2_call(num_rows):
    grid = pl.cdiv(num_rows, _C_BUILD)
    return pl.pallas_call(
        _build_table2_body,
        grid=(grid,),
        in_specs=[
            pl.BlockSpec((_NT, _D), lambda i: (0, 0)),
            pl.BlockSpec((_D, _C_BUILD), lambda i: (0, i)),
        ],
        out_specs=pl.BlockSpec((_C_BUILD, _OUT_W), lambda i: (i, 0)),
        out_shape=jax.ShapeDtypeStruct((num_rows, _OUT_W), jnp.float32),
    )


@functools.lru_cache(maxsize=None)
def _gather_call(n_t, n_b):
    assert n_b == _NW * _BB
    mesh = plsc.VectorSubcoreMesh(
        core_axis_name="c", subcore_axis_name="s", num_cores=_NC,
        num_subcores=_NS)

    def body(idxT_hbm, tab2_hbm, out_hbm,
             idxT_v, buf0, buf1, tb0, tb1, g0, g1, w0, w1):
        wid = lax.axis_index("s") * _NC + lax.axis_index("c")
        col0 = wid * _BB
        pltpu.sync_copy(idxT_hbm.at[:, pl.ds(col0, _BB)], idxT_v)
        bufs = (buf0, buf1)
        tbs = (tb0, tb1)
        gsems = (g0, g1)
        wsems = (w0, w1)
        rowidx = [lax.iota(jnp.int32, _L) + _L * g for g in range(_BB // _L)]

        pltpu.async_copy(tab2_hbm.at[idxT_v.at[0]], buf0, g0)

        def pair_body(i, carry):
            for k in range(2):
                t = 2 * i + k
                bufk, tbk = bufs[k], tbs[k]
                gk, wk = gsems[k], wsems[k]
                pltpu.make_async_copy(
                    tab2_hbm.at[pl.ds(0, _BB)], bufk, gk).wait()

                @pl.when(t + 1 < n_t)
                def _():
                    pltpu.async_copy(
                        tab2_hbm.at[idxT_v.at[t + 1]], bufs[1 - k],
                        gsems[1 - k])

                @pl.when(t >= 2)
                def _():
                    pltpu.make_async_copy(
                        tbk, out_hbm.at[pl.ds(0, _BB), pl.ds(0, _BB)],
                        wk).wait()

                def kd_body(kd, carry2):
                    col = jnp.full((_L,), kd, jnp.int32)
                    for g in range(_BB // _L):
                        vals = plsc.load_gather(bufk, [rowidx[g], col])
                        tbk[kd, pl.ds(g * _L, _L)] = vals
                    return carry2

                lax.fori_loop(0, _OUT_W, kd_body, 0)
                pltpu.async_copy(
                    tbk,
                    out_hbm.at[pl.ds(t * _OUT_W, _OUT_W),
                               pl.ds(col0, _BB)],
                    wk)
            return carry

        lax.fori_loop(0, n_t // 2, pair_body, 0)
        pltpu.make_async_copy(
            tb0, out_hbm.at[pl.ds(0, _BB), pl.ds(0, _BB)], w0).wait()
        pltpu.make_async_copy(
            tb1, out_hbm.at[pl.ds(0, _BB), pl.ds(0, _BB)], w1).wait()

    return pl.kernel(
        body,
        out_type=jax.ShapeDtypeStruct((n_t * _OUT_W, n_b), jnp.float32),
        mesh=mesh,
        scratch_types=[
            pltpu.VMEM((n_t, _BB), jnp.int32),
            pltpu.VMEM((_BB, _OUT_W), jnp.float32),
            pltpu.VMEM((_BB, _OUT_W), jnp.float32),
            pltpu.VMEM((_OUT_W, _BB), jnp.float32),
            pltpu.VMEM((_OUT_W, _BB), jnp.float32),
            pltpu.SemaphoreType.DMA,
            pltpu.SemaphoreType.DMA,
            pltpu.SemaphoreType.DMA,
            pltpu.SemaphoreType.DMA,
        ],
        compiler_params=pltpu.CompilerParams(needs_layout_passes=False),
    )


def kernel(actions, action_embed, learned_token):
    b, t = actions.shape
    actionsT = actions.T
    tableT = action_embed.T
    tok = learned_token.reshape(_NT, _D)
    table2 = _build_table2_call(action_embed.shape[0])(tok, tableT)
    out2 = _gather_call(t, b)(actionsT, table2)
    return out2.reshape(t, _NT, _D, b).transpose(3, 0, 1, 2)
